# Initial kernel scaffold; baseline (speedup 1.0000x reference)
import jax
import jax.numpy as jnp
from jax.experimental import pallas as pl


def kernel(bev_feats, params):
    raise NotImplementedError("write your pallas kernel here")


# speedup vs baseline: 1.4430x; 1.4430x over previous
import functools

import jax
import jax.numpy as jnp
from jax import lax
from jax.experimental import pallas as pl
from jax.experimental.pallas import tpu as pltpu
from jax.experimental.pallas import tpu_sc as plsc

F32 = jnp.float32
I32 = jnp.int32

B = 4
HW = 16384
NPRE = 500
NSLOT = 512


def _dense_stack(bev_feats, p):
    def conv3x3(x, w, b):
        out = lax.conv_general_dilated(
            x, w, (1, 1), 'SAME', dimension_numbers=('NCHW', 'OIHW', 'NCHW'))
        return out + b[None, :, None, None]

    def bn(x, g, b, eps=1e-5):
        mean = x.mean(axis=(0, 2, 3), keepdims=True)
        var = x.var(axis=(0, 2, 3), keepdims=True)
        return ((x - mean) / jnp.sqrt(var + eps) * g[None, :, None, None]
                + b[None, :, None, None])

    def head_fwd(x, i):
        h = conv3x3(x, p['h%d_w1' % i], p['h%d_b1' % i])
        h = jax.nn.relu(bn(h, p['h%d_g1' % i], p['h%d_be1' % i]))
        return conv3x3(h, p['h%d_w2' % i], p['h%d_b2' % i])

    x = jax.nn.relu(bn(conv3x3(bev_feats, p['sw'], p['sb']), p['sg'], p['sbe']))
    center = head_fwd(x, 0)
    center_z = head_fwd(x, 1)
    dim_log = head_fwd(x, 2)
    rot = head_fwd(x, 3)
    hm = jax.nn.sigmoid(head_fwd(x, 4))
    hmax = lax.reduce_window(hm, -jnp.inf, lax.max, (1, 1, 3, 3),
                             (1, 1, 1, 1), 'SAME')
    heat = hm * (hmax == hm).astype(hm.dtype)
    regs = jnp.concatenate([center, center_z, dim_log, rot], axis=1)
    heat8 = jnp.concatenate(
        [heat.reshape(B, 3, HW), jnp.full((B, 5, HW), -1.0, F32)], axis=1)
    return regs.reshape(B, 8, 128, 128), heat8


def _k4_body(heat_ref, sc_ref, ind_ref, work_ref):
    for t in range(24):
        f0 = t * 2048
        c, off = f0 // HW, f0 % HW
        r, col = f0 // 6144, f0 % 6144
        work_ref[r:r + 1, col:col + 2048] = heat_ref[0, c:c + 1, off:off + 2048]
    fiota = (lax.broadcasted_iota(I32, (8, 6144), 0) * 6144
             + lax.broadcasted_iota(I32, (8, 6144), 1)).astype(F32)
    lane = lax.broadcasted_iota(I32, (1, 1, NSLOT), 2)
    sc_ref[...] = jnp.full((1, 1, NSLOT), -1e9, F32)
    ind_ref[...] = jnp.zeros((1, 1, NSLOT), I32)

    def body(i, _):
        wv = work_ref[...]
        m = jnp.max(wv)
        fi = jnp.min(jnp.where(wv == m, fiota, 1e9))
        sc_ref[...] = jnp.where(lane == i, m, sc_ref[...])
        ind_ref[...] = jnp.where(lane == i, fi.astype(I32), ind_ref[...])
        work_ref[...] = jnp.where(fiota == fi, -1.0, wv)
        return 0

    lax.fori_loop(0, NPRE, body, 0)


def _sc_gather(regs, inds):
    info = plsc.get_sparse_core_info()
    nc = info.num_cores
    mesh = plsc.VectorSubcoreMesh(core_axis_name="c", subcore_axis_name="s")

    @functools.partial(
        pl.kernel,
        out_type=jax.ShapeDtypeStruct((B, 8, NSLOT), F32),
        mesh=mesh,
        scratch_types=[
            pltpu.VMEM((128, 128), F32),
            pltpu.VMEM((NSLOT,), I32),
            pltpu.VMEM((NSLOT,), F32),
        ],
        compiler_params=pltpu.CompilerParams(needs_layout_passes=False),
    )
    def k(regs_hbm, inds_hbm, out_hbm, plane_v, idx_v, row_v):
        wid = lax.axis_index("s") * nc + lax.axis_index("c")
        b = wid // 8
        ch = wid % 8
        pltpu.sync_copy(inds_hbm.at[b], idx_v)
        pltpu.sync_copy(regs_hbm.at[b, ch], plane_v)
        for t in range(NSLOT // 16):
            ind16 = idx_v[pl.ds(t * 16, 16)]
            spat16 = jnp.bitwise_and(ind16, 16383)
            r16 = lax.shift_right_logical(spat16, 7)
            c16 = jnp.bitwise_and(spat16, 127)
            row_v[pl.ds(t * 16, 16)] = plsc.load_gather(plane_v, [r16, c16])
        pltpu.sync_copy(row_v, out_hbm.at[b, ch])

    return k(regs, inds)


_DOT = (((1,), (0,)), ((), ()))


def _k6_body(m_ref, sc_ref, vc_ref, sup_ref, out_ref, k_ref):
    sc = sc_ref[0]
    vc = vc_ref[0]
    k_ref[...] = jnp.ones((8, NSLOT), F32)
    lane8 = lax.broadcasted_iota(I32, (8, NSLOT), 1)

    def scan_body(i, _):
        srow = sup_ref[0, pl.ds(i, 1), :]
        kv = k_ref[...]
        gate = jnp.sum(jnp.where(lane8 == i, kv * vc, 0.0), axis=1,
                       keepdims=True)
        k_ref[...] = kv * (1.0 - gate * srow)
        return 0

    lax.fori_loop(0, NSLOT, scan_body, 0)

    keep = jnp.max(k_ref[...] * vc, axis=0, keepdims=True)
    fs0 = jnp.where(keep > 0.5, sc, 0.0)

    ii = (lax.broadcasted_iota(I32, (NSLOT, NSLOT), 0)
          == lax.broadcasted_iota(I32, (NSLOT, NSLOT), 1)).astype(F32)
    hi = jax.lax.Precision.HIGHEST
    _dott = (((1,), (1,)), ((), ()))

    def cols(rows):
        return lax.dot_general(ii, rows, _dott, precision=hi,
                               preferred_element_type=F32)
    lanef = lax.broadcasted_iota(I32, (1, NSLOT), 1).astype(F32)
    lanei = lax.broadcasted_iota(I32, (1, NSLOT), 1)

    def sel_body(i, carry):
        fs, ordv = carry
        m = jnp.max(fs)
        fi = jnp.min(jnp.where(fs == m, lanef, 1e9))
        ordv = jnp.where(lanei == i, fi, ordv)
        fs = jnp.where(lanef == fi, -1.0, fs)
        return fs, ordv

    _, ordv = lax.fori_loop(0, NPRE, sel_body,
                            (fs0, jnp.zeros((1, NSLOT), F32)))
    ordc = cols(ordv)
    perm = (ordc == lanef).astype(F32)
    m = m_ref[0]
    rows16 = jnp.concatenate(
        [m[0:7], fs0, m[8:9], jnp.zeros((7, NSLOT), F32)], axis=0)
    mat16 = cols(rows16)
    res = lax.dot_general(perm, mat16, _DOT, precision=hi,
                          preferred_element_type=F32)
    mask = (res[:, 7:8] > 0.0).astype(F32)
    out_ref[0] = jnp.concatenate(
        [res[:, 0:7] * mask, res[:, 7:16]], axis=1)


def kernel(bev_feats, params):
    regs, heat8 = _dense_stack(bev_feats, params)

    arb1 = pltpu.CompilerParams(dimension_semantics=("arbitrary",))
    scores3, inds3 = pl.pallas_call(
        _k4_body,
        grid=(B,),
        in_specs=[pl.BlockSpec((1, 8, HW), lambda b: (b, 0, 0))],
        out_specs=[
            pl.BlockSpec((1, 1, NSLOT), lambda b: (b, 0, 0)),
            pl.BlockSpec((1, 1, NSLOT), lambda b: (b, 0, 0)),
        ],
        out_shape=[
            jax.ShapeDtypeStruct((B, 1, NSLOT), F32),
            jax.ShapeDtypeStruct((B, 1, NSLOT), I32),
        ],
        scratch_shapes=[pltpu.VMEM((8, 6144), F32)],
        compiler_params=arb1,
    )(heat8)

    feats = _sc_gather(regs, inds3.reshape(B, NSLOT))

    scores = scores3.reshape(B, NSLOT)
    inds = inds3.reshape(B, NSLOT)
    spat = inds % HW
    clsi = inds // HW
    ys = (spat // 128).astype(F32)
    xs = (spat % 128).astype(F32)
    ctr0, ctr1, cz = feats[:, 0], feats[:, 1], feats[:, 2]
    dims = jnp.exp(feats[:, 3:6])
    ang = jnp.arctan2(feats[:, 7], feats[:, 6])
    xs = (xs + ctr0) * 8 * 0.1 + 0.0
    ys = (ys + ctr1) * 8 * 0.1 + (-40.0)
    x1 = xs - dims[:, 0] / 2.0
    y1 = ys - dims[:, 1] / 2.0
    b2d = jnp.stack([x1, y1, x1 + dims[:, 0], y1 + dims[:, 1]], axis=-1)
    lt = jnp.maximum(b2d[:, :, None, :2], b2d[:, None, :, :2])
    rb = jnp.minimum(b2d[:, :, None, 2:], b2d[:, None, :, 2:])
    wh = jnp.clip(rb - lt, 0.0, None)
    inter = wh[..., 0] * wh[..., 1]
    area = (b2d[..., 2] - b2d[..., 0]) * (b2d[..., 3] - b2d[..., 1])
    union = area[:, :, None] + area[:, None, :] - inter
    iou = inter / jnp.maximum(union, 1e-6)
    idx = jnp.arange(NSLOT)
    sup = ((iou > 0.1) & (idx[None, None, :] > idx[None, :, None])) \
        .astype(F32)
    pr = jnp.asarray((-10.0, -50.0, -10.0, 80.4, 50.0, 10.0), F32)
    c3 = jnp.stack([xs, ys, cz], axis=-1)
    rmask = (jnp.all(c3 >= pr[:3], axis=-1) & jnp.all(c3 <= pr[3:], axis=-1))
    valid = (scores > 0.1) & rmask
    r8 = jnp.arange(8)
    vc = ((clsi[:, None, :] == r8[None, :, None]) & valid[:, None, :]
          & (r8[None, :, None] < 3)).astype(F32)
    mat = jnp.stack(
        [xs, ys, cz, dims[:, 0], dims[:, 1], dims[:, 2], ang,
         jnp.zeros_like(xs), clsi.astype(F32)] + [jnp.zeros_like(xs)] * 7,
        axis=1)

    out = pl.pallas_call(
        _k6_body,
        grid=(B,),
        in_specs=[
            pl.BlockSpec((1, 16, NSLOT), lambda b: (b, 0, 0)),
            pl.BlockSpec((1, 1, NSLOT), lambda b: (b, 0, 0)),
            pl.BlockSpec((1, 8, NSLOT), lambda b: (b, 0, 0)),
            pl.BlockSpec((1, NSLOT, NSLOT), lambda b: (b, 0, 0)),
        ],
        out_specs=pl.BlockSpec((1, NSLOT, 16), lambda b: (b, 0, 0)),
        out_shape=jax.ShapeDtypeStruct((B, NSLOT, 16), F32),
        scratch_shapes=[pltpu.VMEM((8, NSLOT), F32)],
        compiler_params=arb1,
    )(mat, scores3, vc, sup)

    rois = out[:, :NPRE, 0:7]
    roi_scores = out[:, :NPRE, 7]
    roi_labels = out[:, :NPRE, 8].astype(I32)
    return rois, roi_scores, roi_labels
